# baseline (speedup 1.0000x reference)
import jax
import jax.numpy as jnp
from jax.experimental import pallas as pl
from jax.experimental.pallas import tpu as pltpu

_NUM_EXPERTS = 8
_H = 1024
_I = 1024
_T = 64
_IC = 512


def _ffn_body(idx_ref, x_ref, g0_ref, g1_ref, u0_ref, u1_ref,
              d0_ref, d1_ref, o_ref):
    e = pl.program_id(0)

    @pl.when(e == 0)
    def _init():
        o_ref[...] = jnp.zeros_like(o_ref)

    mask = idx_ref[...] == e
    xm = jnp.where(mask, x_ref[...], 0.0)
    acc = o_ref[...]
    for g_ref, u_ref, d_ref in ((g0_ref, u0_ref, d0_ref),
                                (g1_ref, u1_ref, d1_ref)):
        g = jnp.dot(xm, g_ref[0], preferred_element_type=jnp.float32)
        u = jnp.dot(xm, u_ref[0], preferred_element_type=jnp.float32)
        inter = g * jax.nn.sigmoid(g) * u
        acc += jax.lax.dot_general(
            inter, d_ref[0], (((1,), (1,)), ((), ())),
            preferred_element_type=jnp.float32)
    o_ref[...] = acc


def kernel(x, expert_indices, gate_proj, up_proj, down_proj):
    b, s, h = x.shape
    x_flat = x.reshape(-1, h)
    idx = expert_indices.reshape(-1, 1).astype(jnp.int32)

    half = pl.BlockSpec((1, _H, _IC), lambda e: (e, 0, 0))
    half_hi = pl.BlockSpec((1, _H, _IC), lambda e: (e, 0, 1))

    out = pl.pallas_call(
        _ffn_body,
        grid=(_NUM_EXPERTS,),
        in_specs=[
            pl.BlockSpec((_T, 1), lambda e: (0, 0)),
            pl.BlockSpec((_T, _H), lambda e: (0, 0)),
            half, half_hi,
            half, half_hi,
            half, half_hi,
        ],
        out_specs=pl.BlockSpec((_T, _H), lambda e: (0, 0)),
        out_shape=jax.ShapeDtypeStruct((_T, _H), jnp.float32),
        compiler_params=pltpu.CompilerParams(
            dimension_semantics=("arbitrary",),
        ),
    )(idx, x_flat, gate_proj, gate_proj, up_proj, up_proj,
      down_proj, down_proj)
    return out.reshape(b, s, h)
